# baseline (speedup 1.0000x reference)
import functools

import jax
import jax.numpy as jnp
from jax import lax
from jax.experimental import pallas as pl
from jax.experimental.pallas import tpu as pltpu
from jax.experimental.pallas import tpu_sc as plsc

N_EMBD = 64
CODEBOOK = 100000
G = 8

NC, NS = 2, 16
NW = NC * NS
LB = 128
CB = 2
TOK = CB * LB
NBUF = 2
L = 16


def kernel(idx, tables):
    B, T, g_dim = idx.shape
    BT = B // LB
    n_blocks = BT * T // CB
    per_w = n_blocks // NW
    EH, EL = N_EMBD // 8, 8

    idx_w = (idx.transpose(1, 2, 0)
                .reshape(T, g_dim, BT, LB)
                .transpose(0, 2, 1, 3))

    mesh = plsc.VectorSubcoreMesh(core_axis_name="c", subcore_axis_name="s")

    @functools.partial(
        pl.kernel,
        out_type=jax.ShapeDtypeStruct((T, EH, BT, EL, LB), jnp.float32),
        mesh=mesh,
        compiler_params=pltpu.CompilerParams(use_tc_tiling_on_sc=False,
                                             needs_layout_passes=False),
        scratch_types=[
            pltpu.VMEM((NBUF, CB, G, LB), jnp.int32),
            pltpu.VMEM((NBUF, G, TOK), jnp.int32),
            pltpu.VMEM((NBUF, TOK, N_EMBD), jnp.float32),
            pltpu.VMEM((NBUF, EH, CB, EL, LB), jnp.float32),
            pltpu.SemaphoreType.DMA((NBUF,)),
            pltpu.SemaphoreType.DMA((NBUF,)),
            pltpu.SemaphoreType.DMA((NBUF,)),
        ],
    )
    def body(idx_hbm, tab_hbm, out_hbm, raw_v, idx_v, acc_v, acct_v,
             sem_idx, sem_acc, sem_out):
        wid = lax.axis_index("s") * NC + lax.axis_index("c")
        blk0 = wid * per_w
        lane = lax.iota(jnp.int32, L)

        def zero_acc(b):
            @pl.loop(0, TOK)
            def _(r):
                for k in range(N_EMBD // L):
                    acc_v[b, r, pl.ds(k * L, L)] = jnp.zeros(
                        (L,), jnp.float32)

        def regroup_idx(b):
            for g in range(G):
                for c in range(CB):
                    for k in range(LB // L):
                        idx_v[b, g, pl.ds(c * LB + k * L, L)] = \
                            raw_v[b, c, g, pl.ds(k * L, L)]

        def fire_gathers(b):
            for g in range(G):
                pltpu.async_copy(tab_hbm.at[g].at[idx_v.at[b, g]],
                                 acc_v.at[b], sem_acc.at[b], add=True)

        def drain_gathers(b):
            for g in range(G):
                pltpu.make_async_copy(tab_hbm.at[g].at[idx_v.at[b, g]],
                                      acc_v.at[b], sem_acc.at[b]).wait()

        def transpose_acc(b):
            @pl.loop(0, EH)
            def _(eh):
                for el in range(EL):
                    col = jnp.full((L,), 0, jnp.int32) + (eh * EL + el)
                    for c in range(CB):
                        for k in range(LB // L):
                            rows = lane + (c * LB + k * L)
                            vals = plsc.load_gather(acc_v.at[b],
                                                    [rows, col])
                            acct_v[b, eh, c, el, pl.ds(k * L, L)] = vals

        def copy_out(b, blk):
            t = blk // (BT // CB)
            bt = (blk - t * (BT // CB)) * CB
            return pltpu.async_copy(acct_v.at[b],
                                    out_hbm.at[t, :, pl.ds(bt, CB)],
                                    sem_out.at[b])

        def load_idx(b, blk, sem=None):
            t = blk // (BT // CB)
            bt = (blk - t * (BT // CB)) * CB
            src = idx_hbm.at[t, pl.ds(bt, CB)]
            if sem is None:
                pltpu.sync_copy(src, raw_v.at[b])
                return None
            return pltpu.async_copy(src, raw_v.at[b], sem.at[b])

        for b in range(NBUF):
            zero_acc(b)
            load_idx(b, blk0 + b)
            regroup_idx(b)
            fire_gathers(b)

        def wait_out(b):
            pltpu.make_async_copy(acct_v.at[b],
                                  out_hbm.at[0, :, pl.ds(0, CB)],
                                  sem_out.at[b]).wait()

        @pl.loop(0, per_w - NBUF - 1, step=NBUF)
        def _(jj):
            for b in range(NBUF):
                blk = blk0 + jj + b
                drain_gathers(b)
                idx_cp = load_idx(b, blk + NBUF, sem_idx)
                @pl.when(jj > 0)
                def _wait_prev_out():
                    wait_out(b)
                transpose_acc(b)
                copy_out(b, blk)
                zero_acc(b)
                idx_cp.wait()
                regroup_idx(b)
                fire_gathers(b)

        last = blk0 + per_w - 1
        drain_gathers(0)
        wait_out(0)
        transpose_acc(0)
        copy_out(0, last - 2)
        zero_acc(0)
        load_idx(0, last)
        regroup_idx(0)
        fire_gathers(0)

        drain_gathers(1)
        wait_out(1)
        transpose_acc(1)
        out_cp1 = copy_out(1, last - 1)

        drain_gathers(0)
        wait_out(0)
        transpose_acc(0)
        copy_out(0, last).wait()
        out_cp1.wait()

    out5 = body(idx_w, tables)
    return (out5.transpose(2, 4, 0, 1, 3)
                .reshape(B, T, N_EMBD))


# speedup vs baseline: 1.1087x; 1.1087x over previous
import functools

import jax
import jax.numpy as jnp
from jax import lax
from jax.experimental import pallas as pl
from jax.experimental.pallas import tpu as pltpu
from jax.experimental.pallas import tpu_sc as plsc

N_EMBD = 64
CODEBOOK = 100000
G = 8

NC, NS = 2, 16
NW = NC * NS
LB = 128
CB = 2
TOK = CB * LB
NBUF = 2
L = 16


def kernel(idx, tables):
    B, T, g_dim = idx.shape
    BT = B // LB
    bpt = BT // CB
    n_blocks = bpt * T
    per_w = n_blocks // NW

    idx_w = (idx.transpose(1, 2, 0)
                .reshape(T, g_dim, BT, LB)
                .transpose(0, 2, 1, 3))

    mesh = plsc.VectorSubcoreMesh(core_axis_name="c", subcore_axis_name="s")

    @functools.partial(
        pl.kernel,
        out_type=jax.ShapeDtypeStruct((B, T, N_EMBD), jnp.float32),
        mesh=mesh,
        compiler_params=pltpu.CompilerParams(use_tc_tiling_on_sc=False,
                                             needs_layout_passes=False),
        scratch_types=[
            pltpu.VMEM((NBUF, CB, G, LB), jnp.int32),
            pltpu.VMEM((NBUF, G, TOK), jnp.int32),
            pltpu.VMEM((NBUF, TOK, N_EMBD), jnp.float32),
            pltpu.SemaphoreType.DMA((NBUF,)),
            pltpu.SemaphoreType.DMA((NBUF,)),
            pltpu.SemaphoreType.DMA((NBUF,)),
        ],
    )
    def body(idx_hbm, tab_hbm, out_hbm,
             raw_v, idx_v, acc_v, sem_idx, sem_acc, sem_out):
        wid = lax.axis_index("s") * NC + lax.axis_index("c")
        blk0 = wid * per_w

        def zero_acc(b):
            @pl.loop(0, TOK)
            def _(r):
                for k in range(N_EMBD // L):
                    acc_v[b, r, pl.ds(k * L, L)] = jnp.zeros(
                        (L,), jnp.float32)

        def regroup_idx(b):
            for g in range(G):
                for c in range(CB):
                    for k in range(LB // L):
                        idx_v[b, g, pl.ds(c * LB + k * L, L)] = \
                            raw_v[b, c, g, pl.ds(k * L, L)]

        def fire_gathers(b):
            for g in range(G):
                pltpu.async_copy(tab_hbm.at[g].at[idx_v.at[b, g]],
                                 acc_v.at[b], sem_acc.at[b], add=True)

        def drain_gathers(b):
            for g in range(G):
                pltpu.make_async_copy(tab_hbm.at[g].at[idx_v.at[b, g]],
                                      acc_v.at[b], sem_acc.at[b]).wait()

        def copy_out(b, blk):
            t = blk // bpt
            b0 = (blk - t * bpt) * TOK
            return pltpu.async_copy(acc_v.at[b],
                                    out_hbm.at[pl.ds(b0, TOK), t],
                                    sem_out.at[b])

        def load_idx(b, blk, sem=None):
            t = blk // bpt
            bt = (blk - t * bpt) * CB
            src = idx_hbm.at[t, pl.ds(bt, CB)]
            if sem is None:
                pltpu.sync_copy(src, raw_v.at[b])
                return None
            return pltpu.async_copy(src, raw_v.at[b], sem.at[b])

        for b in range(NBUF):
            zero_acc(b)
            load_idx(b, blk0 + b)
            regroup_idx(b)
            fire_gathers(b)

        @pl.loop(0, per_w - NBUF - 1, step=NBUF)
        def _(jj):
            for b in range(NBUF):
                blk = blk0 + jj + b
                drain_gathers(b)
                idx_cp = load_idx(b, blk + NBUF, sem_idx)
                copy_out(b, blk).wait()
                zero_acc(b)
                idx_cp.wait()
                regroup_idx(b)
                fire_gathers(b)

        last = blk0 + per_w - 1
        drain_gathers(0)
        copy_out(0, last - 2).wait()
        zero_acc(0)
        load_idx(0, last)
        regroup_idx(0)
        fire_gathers(0)

        drain_gathers(1)
        copy_out(1, last - 1).wait()

        drain_gathers(0)
        copy_out(0, last).wait()

    return body(idx_w, tables)
